# baseline (speedup 1.0000x reference)
import jax
import jax.numpy as jnp
from jax import lax
from jax.experimental import pallas as pl
from jax.experimental.pallas import tpu as pltpu
from jax.experimental.pallas import tpu_sc as plsc

LANES = 16
BLK = 100
JPC = 2
NC = 1
NS = 16
NW = NC * NS


def _body(x_hbm, tok_hbm, pos_hbm, out_hbm,
          idx_all, rows_v, pos_v, gsem0, gsem1, osem0, osem1):
    emb = tok_hbm.shape[1]
    wid = lax.axis_index("s") * NC + lax.axis_index("c")
    nblocks = x_hbm.shape[0]
    blocks_w = nblocks // NW
    nchunks = blocks_w // JPC
    gsems = (gsem0, gsem1)
    osems = (osem0, osem1)

    pltpu.sync_copy(x_hbm.at[pl.ds(wid * blocks_w, blocks_w)], idx_all)
    pltpu.sync_copy(pos_hbm, pos_v)

    def start_gather(c, b):
        for j in range(JPC):
            pltpu.async_copy(
                tok_hbm.at[idx_all.at[c * JPC + j]], rows_v.at[b, j], gsems[b])

    start_gather(0, 0)

    def outer(g, carry):
        for b in range(2):
            c = g * 2 + b

            @pl.when(c >= 1)
            def _():
                pltpu.make_async_copy(
                    rows_v.at[1 - b], out_hbm.at[pl.ds(0, JPC)],
                    osems[1 - b]).wait()

            @pl.when(c + 1 < nchunks)
            def _():
                start_gather(c + 1, 1 - b)

            pltpu.make_async_copy(
                out_hbm.at[pl.ds(0, JPC)], rows_v.at[b], gsems[b]).wait()

            def add_body(r, cr):
                for j in range(JPC):
                    for k in range(emb // LANES):
                        sl = pl.ds(k * LANES, LANES)
                        plsc.addupdate(rows_v.at[b, j, r, sl], pos_v[j, r, sl])
                return cr

            lax.fori_loop(0, BLK, add_body, 0, unroll=4)

            row = wid * blocks_w + c * JPC
            pltpu.async_copy(rows_v.at[b], out_hbm.at[pl.ds(row, JPC)],
                             osems[b])
        return carry

    lax.fori_loop(0, nchunks // 2, outer, 0)

    pltpu.make_async_copy(
        rows_v.at[1], out_hbm.at[pl.ds(0, JPC)], osems[1]).wait()


def kernel(x, tok_table, pos_table):
    b, s = x.shape
    v, e = tok_table.shape
    n = b * s
    assert s == JPC * BLK and e % LANES == 0
    assert n % (NW * 2 * JPC * BLK) == 0

    x2 = x.reshape(n // BLK, BLK)
    pos2 = pos_table[:s].reshape(JPC, BLK, e)

    out = pl.kernel(
        _body,
        out_type=jax.ShapeDtypeStruct((n // BLK, BLK, e), jnp.float32),
        mesh=plsc.VectorSubcoreMesh(core_axis_name="c", subcore_axis_name="s", num_cores=NC),
        compiler_params=pltpu.CompilerParams(use_tc_tiling_on_sc=False),
        scratch_types=[
            pltpu.VMEM((n // BLK // NW, BLK), jnp.int32),
            pltpu.VMEM((2, JPC, BLK, e), jnp.float32),
            pltpu.VMEM((JPC, BLK, e), jnp.float32),
            pltpu.SemaphoreType.DMA,
            pltpu.SemaphoreType.DMA,
            pltpu.SemaphoreType.DMA,
            pltpu.SemaphoreType.DMA,
        ],
    )(x2, tok_table, pos2)
    return out.reshape(b, s, e)


# speedup vs baseline: 1.0087x; 1.0087x over previous
import jax
import jax.numpy as jnp
from jax import lax
from jax.experimental import pallas as pl
from jax.experimental.pallas import tpu as pltpu
from jax.experimental.pallas import tpu_sc as plsc

LANES = 16
CHB = 128
NC = 2
NS = 16
NW = NC * NS
BSPLIT = 4
SSPLIT = NW // BSPLIT


def _body(x_hbm, tok_hbm, pos_hbm, out_hbm,
          idx_all, pos_v, pair_v, poff_v, rows_v, out_v,
          gsem0, gsem1, osem0, osem1):
    s_len, batch = x_hbm.shape
    emb = pos_hbm.shape[1]
    wid = lax.axis_index("s") * NC + lax.axis_index("c")
    su = wid // BSPLIT
    bv = lax.rem(wid, BSPLIT)
    srows = s_len // SSPLIT
    bcols = batch // BSPLIT
    cpr = bcols // CHB
    nchunks = srows * cpr
    gsems = (gsem0, gsem1)
    osems = (osem0, osem1)

    roff = su * (srows - 1)
    pltpu.sync_copy(
        x_hbm.at[pl.ds(roff, srows + 7), pl.ds(bv * bcols, bcols)], idx_all)
    pltpu.sync_copy(pos_hbm, pos_v)

    def prep(c, b):
        for m in range(CHB // LANES):
            sl = pl.ds(m * LANES, LANES)
            v = idx_all[su + c // cpr,
                        pl.ds(lax.rem(c, cpr) * CHB + m * LANES, LANES)]
            pair_v[b, sl] = lax.shift_right_logical(v, 1)
            poff_v[b, sl] = lax.shift_left(lax.bitwise_and(v, 1), 6)

    def start_gather(b):
        pltpu.async_copy(tok_hbm.at[pair_v.at[b]], rows_v.at[b], gsems[b])

    prep(0, 0)
    start_gather(0)

    def outer(g, carry):
        for b in range(2):
            c = g * 2 + b

            @pl.when(c >= 2)
            def _():
                pltpu.make_async_copy(
                    out_v.at[b], out_hbm.at[0, pl.ds(0, CHB), :],
                    osems[b]).wait()

            @pl.when(c + 1 < nchunks)
            def _():
                prep(c + 1, 1 - b)
                start_gather(1 - b)

            pltpu.make_async_copy(
                tok_hbm.at[pl.ds(0, CHB)], rows_v.at[b], gsems[b]).wait()

            s = su * srows + c // cpr
            pos_regs = [pos_v[s, pl.ds(k * LANES, LANES)]
                        for k in range(emb // LANES)]
            for tg in range(CHB // LANES):
                pvec = poff_v[b, pl.ds(tg * LANES, LANES)]
                for l in range(LANES):
                    t = tg * LANES + l
                    po = pvec[l]
                    for k in range(emb // LANES):
                        val = rows_v[b, t, pl.ds(po + k * LANES, LANES)]
                        out_v[b, t, pl.ds(k * LANES, LANES)] = (
                            val + pos_regs[k])

            b0 = bv * bcols + lax.rem(c, cpr) * CHB
            pltpu.async_copy(out_v.at[b], out_hbm.at[s, pl.ds(b0, CHB), :],
                             osems[b])
        return carry

    lax.fori_loop(0, nchunks // 2, outer, 0)

    for b in range(2):
        pltpu.make_async_copy(
            out_v.at[b], out_hbm.at[0, pl.ds(0, CHB), :], osems[b]).wait()


def kernel(x, tok_table, pos_table):
    bsz, s_len = x.shape
    v, e = tok_table.shape
    assert v % 2 == 0 and e % LANES == 0
    assert s_len % SSPLIT == 0 and bsz % (BSPLIT * CHB) == 0
    assert (s_len // SSPLIT) * (bsz // (BSPLIT * CHB)) % 2 == 0

    xT = x.T
    tok2 = tok_table.reshape(v // 2, 2 * e)

    outT = pl.kernel(
        _body,
        out_type=jax.ShapeDtypeStruct((s_len, bsz, e), jnp.float32),
        mesh=plsc.VectorSubcoreMesh(core_axis_name="c", subcore_axis_name="s",
                                    num_cores=NC),
        compiler_params=pltpu.CompilerParams(use_tc_tiling_on_sc=True),
        scratch_types=[
            pltpu.VMEM((s_len // SSPLIT + 7, bsz // BSPLIT), jnp.int32),
            pltpu.VMEM((s_len, e), jnp.float32),
            pltpu.VMEM((2, CHB), jnp.int32),
            pltpu.VMEM((2, CHB), jnp.int32),
            pltpu.VMEM((2, CHB, 2 * e), jnp.float32),
            pltpu.VMEM((2, CHB, e), jnp.float32),
            pltpu.SemaphoreType.DMA,
            pltpu.SemaphoreType.DMA,
            pltpu.SemaphoreType.DMA,
            pltpu.SemaphoreType.DMA,
        ],
    )(xT, tok2, pos_table)
    return outT.transpose(1, 0, 2)


# speedup vs baseline: 1.0579x; 1.0487x over previous
import jax
import jax.numpy as jnp
from jax import lax
from jax.experimental import pallas as pl
from jax.experimental.pallas import tpu as pltpu
from jax.experimental.pallas import tpu_sc as plsc

LANES = 16
BLK = 100
JPC = 2
NC = 2
NS = 16
NW = NC * NS


def _body(x_hbm, tok_hbm, pos_hbm, out_hbm,
          idx_all, rows_v, pos_v, gsem0, gsem1, osem0, osem1):
    emb = tok_hbm.shape[1]
    wid = lax.axis_index("s") * NC + lax.axis_index("c")
    nblocks = x_hbm.shape[0]
    blocks_w = nblocks // NW
    nchunks = blocks_w // JPC
    gsems = (gsem0, gsem1)
    osems = (osem0, osem1)

    pltpu.sync_copy(x_hbm.at[pl.ds(wid * blocks_w, blocks_w)], idx_all)
    pltpu.sync_copy(pos_hbm, pos_v)

    def start_gather(c, b):
        for j in range(JPC):
            pltpu.async_copy(
                tok_hbm.at[idx_all.at[c * JPC + j]], rows_v.at[b, j], gsems[b])

    start_gather(0, 0)

    def outer(g, carry):
        for b in range(2):
            c = g * 2 + b

            @pl.when(c >= 1)
            def _():
                pltpu.make_async_copy(
                    rows_v.at[1 - b], out_hbm.at[pl.ds(0, JPC)],
                    osems[1 - b]).wait()

            @pl.when(c + 1 < nchunks)
            def _():
                start_gather(c + 1, 1 - b)

            pltpu.make_async_copy(
                out_hbm.at[pl.ds(0, JPC)], rows_v.at[b], gsems[b]).wait()

            def add_body(r, cr):
                for j in range(JPC):
                    for k in range(emb // LANES):
                        sl = pl.ds(k * LANES, LANES)
                        plsc.addupdate(rows_v.at[b, j, r, sl], pos_v[j, r, sl])
                return cr

            lax.fori_loop(0, BLK, add_body, 0, unroll=4)

            row = wid * blocks_w + c * JPC
            pltpu.async_copy(rows_v.at[b], out_hbm.at[pl.ds(row, JPC)],
                             osems[b])
        return carry

    lax.fori_loop(0, nchunks // 2, outer, 0)

    pltpu.make_async_copy(
        rows_v.at[1], out_hbm.at[pl.ds(0, JPC)], osems[1]).wait()


def kernel(x, tok_table, pos_table):
    b, s = x.shape
    v, e = tok_table.shape
    n = b * s
    assert s == JPC * BLK and e % LANES == 0
    assert n % (NW * 2 * JPC * BLK) == 0

    x2 = x.reshape(n // BLK, BLK)
    pos2 = pos_table[:s].reshape(JPC, BLK, e)

    out = pl.kernel(
        _body,
        out_type=jax.ShapeDtypeStruct((n // BLK, BLK, e), jnp.float32),
        mesh=plsc.VectorSubcoreMesh(core_axis_name="c", subcore_axis_name="s", num_cores=NC),
        compiler_params=pltpu.CompilerParams(use_tc_tiling_on_sc=False),
        scratch_types=[
            pltpu.VMEM((n // BLK // NW, BLK), jnp.int32),
            pltpu.VMEM((2, JPC, BLK, e), jnp.float32),
            pltpu.VMEM((JPC, BLK, e), jnp.float32),
            pltpu.SemaphoreType.DMA,
            pltpu.SemaphoreType.DMA,
            pltpu.SemaphoreType.DMA,
            pltpu.SemaphoreType.DMA,
        ],
    )(x2, tok_table, pos2)
    return out.reshape(b, s, e)


# speedup vs baseline: 1.0618x; 1.0038x over previous
import jax
import jax.numpy as jnp
from jax import lax
from jax.experimental import pallas as pl
from jax.experimental.pallas import tpu as pltpu
from jax.experimental.pallas import tpu_sc as plsc

LANES = 16
CHB = 128
NC = 2
NS = 16
NW = NC * NS
BSPLIT = 4
SSPLIT = NW // BSPLIT


def _body(x_hbm, tok_hbm, pos_hbm, out_hbm,
          idx_all, pos_v, rows_v, gsem0, gsem1, osem0, osem1):
    s_len, batch = x_hbm.shape
    emb = pos_hbm.shape[1]
    wid = lax.axis_index("s") * NC + lax.axis_index("c")
    su = wid // BSPLIT
    bv = lax.rem(wid, BSPLIT)
    srows = s_len // SSPLIT
    bcols = batch // BSPLIT
    cpr = bcols // CHB
    nchunks = srows * cpr
    gsems = (gsem0, gsem1)
    osems = (osem0, osem1)

    roff = su * (srows - 1)
    pltpu.sync_copy(
        x_hbm.at[pl.ds(roff, srows + 7), pl.ds(bv * bcols, bcols)], idx_all)
    pltpu.sync_copy(pos_hbm, pos_v)

    def start_gather(c, b):
        idx = idx_all.at[su + c // cpr, pl.ds(lax.rem(c, cpr) * CHB, CHB)]
        pltpu.async_copy(tok_hbm.at[idx], rows_v.at[b], gsems[b])

    start_gather(0, 0)

    def outer(g, carry):
        for b in range(2):
            c = g * 2 + b

            @pl.when(c >= 1)
            def _():
                pltpu.make_async_copy(
                    rows_v.at[1 - b], out_hbm.at[0, pl.ds(0, CHB), :],
                    osems[1 - b]).wait()

            @pl.when(c + 1 < nchunks)
            def _():
                start_gather(c + 1, 1 - b)

            pltpu.make_async_copy(
                tok_hbm.at[pl.ds(0, CHB)], rows_v.at[b], gsems[b]).wait()

            s = su * srows + c // cpr
            pos_regs = [pos_v[s, pl.ds(k * LANES, LANES)]
                        for k in range(emb // LANES)]
            for t in range(CHB):
                for k in range(emb // LANES):
                    plsc.addupdate(
                        rows_v.at[b, t, pl.ds(k * LANES, LANES)], pos_regs[k])

            b0 = bv * bcols + lax.rem(c, cpr) * CHB
            pltpu.async_copy(rows_v.at[b], out_hbm.at[s, pl.ds(b0, CHB), :],
                             osems[b])
        return carry

    lax.fori_loop(0, nchunks // 2, outer, 0)

    pltpu.make_async_copy(
        rows_v.at[1], out_hbm.at[0, pl.ds(0, CHB), :], osems[1]).wait()


def kernel(x, tok_table, pos_table):
    bsz, s_len = x.shape
    v, e = tok_table.shape
    assert e % LANES == 0
    assert s_len % SSPLIT == 0 and bsz % (BSPLIT * CHB) == 0
    assert (s_len // SSPLIT) * (bsz // (BSPLIT * CHB)) % 2 == 0

    xT = x.T

    outT = pl.kernel(
        _body,
        out_type=jax.ShapeDtypeStruct((s_len, bsz, e), jnp.float32),
        mesh=plsc.VectorSubcoreMesh(core_axis_name="c", subcore_axis_name="s",
                                    num_cores=NC),
        compiler_params=pltpu.CompilerParams(use_tc_tiling_on_sc=False),
        scratch_types=[
            pltpu.VMEM((s_len // SSPLIT + 7, bsz // BSPLIT), jnp.int32),
            pltpu.VMEM((s_len, e), jnp.float32),
            pltpu.VMEM((2, CHB, e), jnp.float32),
            pltpu.SemaphoreType.DMA,
            pltpu.SemaphoreType.DMA,
            pltpu.SemaphoreType.DMA,
            pltpu.SemaphoreType.DMA,
        ],
    )(xT, tok_table, pos_table)
    return outT.transpose(1, 0, 2)
